# baseline (speedup 1.0000x reference)
import functools

import jax
import jax.numpy as jnp
from jax import lax
from jax.experimental import pallas as pl
from jax.experimental.pallas import tpu as pltpu
from jax.experimental.pallas import tpu_sc as plsc

DIM = 64
BATCH = 16384
GAMMA = 12.0
LANES = 16

NUM_CORES = 2
NUM_SUBCORES = 16
NUM_WORKERS = NUM_CORES * NUM_SUBCORES
BPW = BATCH // NUM_WORKERS
GCHUNK = 128
NCHUNK = BPW // GCHUNK

_mesh = plsc.VectorSubcoreMesh(core_axis_name="c", subcore_axis_name="s")


@functools.partial(
    pl.kernel,
    mesh=_mesh,
    out_type=jax.ShapeDtypeStruct((BATCH,), jnp.float32),
    scratch_types=[
        pltpu.VMEM((NCHUNK, GCHUNK), jnp.int32),
        pltpu.VMEM((NCHUNK, GCHUNK), jnp.int32),
        pltpu.VMEM((NCHUNK, GCHUNK), jnp.int32),
        pltpu.VMEM((BPW, DIM), jnp.float32),
        pltpu.VMEM((BPW, DIM), jnp.float32),
        pltpu.VMEM((BPW, DIM), jnp.float32),
        pltpu.VMEM((BPW,), jnp.float32),
        pltpu.SemaphoreType.DMA,
    ],
    compiler_params=pltpu.CompilerParams(use_tc_tiling_on_sc=False),
)
def _score_kernel(hidx_hbm, ridx_hbm, tidx_hbm, ent_hbm, rel_hbm, out_hbm,
                  hidx, ridx, tidx, hrow, rrow, trow, orow, sem):
    wid = lax.axis_index("s") * NUM_CORES + lax.axis_index("c")
    base = wid * BPW

    pltpu.sync_copy(hidx_hbm.at[wid], hidx)
    pltpu.sync_copy(ridx_hbm.at[wid], ridx)
    pltpu.sync_copy(tidx_hbm.at[wid], tidx)

    copies = []
    for j in range(NCHUNK):
        dst = pl.ds(j * GCHUNK, GCHUNK)
        copies.append(pltpu.async_copy(ent_hbm.at[hidx.at[j]], hrow.at[dst], sem))
        copies.append(pltpu.async_copy(rel_hbm.at[ridx.at[j]], rrow.at[dst], sem))
        copies.append(pltpu.async_copy(ent_hbm.at[tidx.at[j]], trow.at[dst], sem))
    for c in copies:
        c.wait()

    lane_iota = lax.iota(jnp.int32, LANES)
    _dnums = lax.GatherDimensionNumbers(
        offset_dims=(), collapsed_slice_dims=(0,), start_index_map=(0,))

    def _lane_perm(v, idx):
        return lax.gather(v, idx[:, None], _dnums, slice_sizes=(1,),
                          unique_indices=True,
                          mode=lax.GatherScatterMode.PROMISE_IN_BOUNDS)

    def group(g, _):
        def row(k, vec):
            i = g * LANES + k
            acc = jnp.zeros((LANES,), jnp.float32)
            for j in range(DIM // LANES):
                s = pl.ds(j * LANES, LANES)
                acc = acc + jnp.abs(hrow[i, s] + rrow[i, s] - trow[i, s])
            for sh in (8, 4, 2, 1):
                acc = acc + _lane_perm(acc, lane_iota ^ sh)
            return jnp.where(lane_iota == k, acc - GAMMA, vec)

        orow[pl.ds(g * LANES, LANES)] = lax.fori_loop(
            0, LANES, row, jnp.zeros((LANES,), jnp.float32))
        return 0

    lax.fori_loop(0, BPW // LANES, group, 0)

    pltpu.sync_copy(orow, out_hbm.at[pl.ds(base, BPW)])


def kernel(pos_sample, ent_embd, rel_embd):
    idx3 = (NUM_WORKERS, NCHUNK, GCHUNK)
    hidx = pos_sample[:, 0].reshape(idx3)
    ridx = pos_sample[:, 1].reshape(idx3)
    tidx = pos_sample[:, 2].reshape(idx3)
    score = _score_kernel(hidx, ridx, tidx, ent_embd, rel_embd)
    return score[:, None]
